# baseline (speedup 1.0000x reference)
import functools

import jax
import jax.numpy as jnp
from jax.experimental import pallas as pl
from jax.experimental.pallas import tpu as pltpu

_B, _V, _S, _D = 64, 100000, 32, 768
_R = 8
_IMIN = -2147483648
_KEY_NEG_INF = -2139095040
_KEY_POS_INF = 2139095040
_TOPK = 50
_P = 0.9


def _order_key(x):
    b = jax.lax.bitcast_convert_type(x, jnp.int32)
    return jnp.where(b >= 0, b, jnp.int32(_IMIN) - b)


def _midpoint(lo, hi):
    return (lo >> 1) + (hi >> 1) + (lo & hi & 1)


def _tree(parts, op):
    while len(parts) > 1:
        nxt = [op(parts[i], parts[i + 1]) for i in range(0, len(parts) - 1, 2)]
        if len(parts) % 2:
            nxt.append(parts[-1])
        parts = nxt
    return parts[0]


def _reduce_wide(x, op, fin, comb):
    a = _tree([x[:, i * 12544:(i + 1) * 12544] for i in range(7)], op)
    a = _tree([a[:, i * 1792:(i + 1) * 1792] for i in range(7)], op)
    tl = _tree([x[:, 87808 + i * 2048: 87808 + (i + 1) * 2048]
                for i in range(5)], op)
    d = x[:, 98048:100000]
    return comb(comb(fin(a), fin(tl)), fin(d))


def _rsum(x):
    return _reduce_wide(x, jnp.add,
                        lambda v: jnp.sum(v, axis=-1, keepdims=True), jnp.add)


def _rmax(x):
    return _reduce_wide(x, jnp.maximum,
                        lambda v: jnp.max(v, axis=-1, keepdims=True),
                        jnp.maximum)


def _rmin(x):
    return _reduce_wide(x, jnp.minimum,
                        lambda v: jnp.min(v, axis=-1, keepdims=True),
                        jnp.minimum)


def _body(t_ref, l_ref, g0_ref, g1_ref, g2_ref, g3_ref,
          out_ref, e_ref, key_ref):
    t = t_ref[0, 0]
    l = l_ref[...] / t
    key = _order_key(l)
    key_ref[...] = key
    m = _rmax(l)
    e = jnp.exp(l - m)
    e_ref[...] = e
    z = _rsum(e)
    pz = jnp.float32(_P) * z

    ones = jnp.ones((_R, 1), dtype=jnp.int32)
    lo0 = ones * _KEY_NEG_INF
    hi0 = ones * _KEY_POS_INF

    def it(_, carry):
        lo_k, hi_k, lo_n, hi_n = carry
        mid_k = _midpoint(lo_k, hi_k)
        mid_n = _midpoint(lo_n, hi_n)
        kk = key_ref[...]
        cnt = _rsum(jnp.where(kk > mid_k, jnp.float32(1.0), jnp.float32(0.0)))
        gs = _rsum(jnp.where(kk > mid_n, e_ref[...], jnp.float32(0.0)))
        big_k = cnt >= jnp.float32(_TOPK)
        lo_k = jnp.where(big_k, mid_k, lo_k)
        hi_k = jnp.where(big_k, hi_k, mid_k)
        big_n = gs > pz
        lo_n = jnp.where(big_n, mid_n, lo_n)
        hi_n = jnp.where(big_n, hi_n, mid_n)
        return lo_k, hi_k, lo_n, hi_n

    lo_k, _, lo_n, _ = jax.lax.fori_loop(0, 32, it, (lo0, hi0, lo0, hi0))

    keep_k = key > lo_k
    keep_n = key > lo_n

    probs = e / z
    logp = jnp.log(probs + jnp.float32(1e-10))
    ent = -_rsum(probs * logp)
    keep_y = jnp.abs(-logp - ent) < jnp.float32(0.5)

    neg_inf = jnp.float32(-jnp.inf)
    iota = jax.lax.broadcasted_iota(jnp.int32, (_R, _V), 1)
    sentinel = jnp.int32(_V)

    def sample(keep, g_ref):
        g = -jnp.log(-jnp.log(g_ref[...]))
        vals = jnp.where(keep, l, neg_inf) + g
        mx = _rmax(vals)
        return _rmin(jnp.where(vals == mx, iota, sentinel))

    s_n = sample(keep_n, g0_ref)
    s_k = sample(keep_k, g1_ref)
    s_t = sample(jnp.ones((_R, _V), dtype=jnp.bool_), g2_ref)
    s_y = sample(keep_y, g3_ref)
    out_ref[...] = jnp.concatenate([s_n, s_k, s_t, s_y], axis=-1)


def _mlp_body(h_ref, w1_ref, b1_ref, w2_ref, b2_ref, s_ref, out_ref):
    h = jnp.mean(h_ref[...], axis=1)
    z1 = jax.nn.relu(
        jnp.dot(h, w1_ref[...], preferred_element_type=jnp.float32)
        + b1_ref[...])
    z2 = (jnp.dot(z1, w2_ref[...], preferred_element_type=jnp.float32)
          + b2_ref[...])
    w = jax.nn.softmax(z2, axis=-1)
    samples = s_ref[...].astype(jnp.float32)
    weighted = jnp.sum(samples * w, axis=-1, keepdims=True)
    out_ref[...] = weighted.astype(jnp.int32)


@functools.partial(jax.jit, static_argnames=())
def kernel(logits, hidden_states, W1, b1, W2, b2, temperature=1.0):
    skey = jax.random.key(42)
    tiny = jnp.finfo(jnp.float32).tiny
    g = [jax.random.uniform(jax.random.fold_in(skey, i), (_B, _V), jnp.float32,
                            minval=tiny, maxval=1.0)
         for i in range(4)]
    t = jnp.asarray(temperature, jnp.float32).reshape(1, 1)
    b1r = b1.reshape(1, 256)
    b2r = b2.reshape(1, 4)

    grid = _B // _R
    row_spec = pl.BlockSpec((_R, _V), lambda i: (i, 0))
    samples = pl.pallas_call(
        _body,
        grid=(grid,),
        in_specs=[
            pl.BlockSpec((1, 1), lambda i: (0, 0)),
            row_spec, row_spec, row_spec, row_spec, row_spec,
        ],
        out_specs=pl.BlockSpec((_R, 4), lambda i: (i, 0)),
        out_shape=jax.ShapeDtypeStruct((_B, 4), jnp.int32),
        scratch_shapes=[
            pltpu.VMEM((_R, _V), jnp.float32),
            pltpu.VMEM((_R, _V), jnp.int32),
        ],
    )(t, logits, g[0], g[1], g[2], g[3])

    out = pl.pallas_call(
        _mlp_body,
        in_specs=[
            pl.BlockSpec((_B, _S, _D), lambda: (0, 0, 0)),
            pl.BlockSpec((_D, 256), lambda: (0, 0)),
            pl.BlockSpec((1, 256), lambda: (0, 0)),
            pl.BlockSpec((256, 4), lambda: (0, 0)),
            pl.BlockSpec((1, 4), lambda: (0, 0)),
            pl.BlockSpec((_B, 4), lambda: (0, 0)),
        ],
        out_specs=pl.BlockSpec((_B, 1), lambda: (0, 0)),
        out_shape=jax.ShapeDtypeStruct((_B, 1), jnp.int32),
    )(hidden_states, W1, b1r, W2, b2r, samples)
    return out.reshape(_B)


# speedup vs baseline: 1.1770x; 1.1770x over previous
import functools

import jax
import jax.numpy as jnp
from jax import lax
from jax.experimental import pallas as pl
from jax.experimental.pallas import tpu as pltpu
from jax.experimental.pallas import tpu_sc as plsc

_B, _V, _S, _D = 64, 100000, 32, 768
_R = 8
_IMIN = -2147483648
_TOPK = 50
_P = 0.9

_NB = 8192
_SH = 19
_CH = 4000
_NCH = _V // _CH
_BUF = 8192


def _order_key(x):
    b = lax.bitcast_convert_type(x, jnp.int32)
    return jnp.where(b >= 0, b, jnp.int32(_IMIN) - b)


def _sc_kernel(l_hbm, t_hbm, out_hbm, stage, hcnt, hsum, bufk, bufn, bufe,
               tv, ob):
    wid = lax.axis_index("s") * 2 + lax.axis_index("c")
    pltpu.sync_copy(t_hbm, tv)
    t = tv[...]
    lanes = lax.iota(jnp.int32, 16)
    onesf = jnp.ones((16,), jnp.float32)
    zerof = jnp.zeros((16,), jnp.float32)

    for rr in range(2):
        row = wid * 2 + rr

        def zbody(i, c):
            hcnt[pl.ds(i * 16, 16)] = zerof
            hsum[pl.ds(i * 16, 16)] = zerof
            return c
        lax.fori_loop(0, _NB // 16, zbody, 0)

        def p1chunk(ci, c):
            pltpu.sync_copy(l_hbm.at[pl.ds(row * _V + ci * _CH, _CH)], stage)

            def p1vec(vi, c2):
                lv = stage[pl.ds(vi * 16, 16)] / t
                key = _order_key(lv)
                biased = key ^ jnp.int32(_IMIN)
                hb = lax.shift_right_logical(biased, _SH)
                ev = jnp.exp(lv)
                plsc.addupdate_scatter(hcnt, [hb], onesf)
                plsc.addupdate_scatter(hsum, [hb], ev)
                return c2
            lax.fori_loop(0, _CH // 16, p1vec, 0)
            return c
        lax.fori_loop(0, _NCH, p1chunk, 0)

        def zsum(i, acc):
            return acc + jnp.sum(hsum[pl.ds(i * 16, 16)])
        ztot = lax.fori_loop(0, _NB // 16, zsum, jnp.float32(0.0))
        pz = jnp.float32(_P) * ztot

        def scan_hist(href, thresh):
            def sbody(i, carry):
                acc, fbin, fcarry, found = carry
                c = (_NB // 16 - 1) - i
                vec = href[pl.ds(c * 16, 16)]
                tot = jnp.sum(vec)
                cross = jnp.logical_and(found == 0, acc + tot > thresh)
                d = lax.rev(vec, (0,))
                cum = plsc.cumsum(d)
                mask = (acc + cum) > thresh
                j = jnp.max(plsc.all_reduce_ffs(mask))
                cumprev = cum - d
                catj = acc + jnp.sum(jnp.where(lanes == j, cumprev, zerof))
                binj = c * 16 + 15 - j
                fbin = jnp.where(cross, binj, fbin)
                fcarry = jnp.where(cross, catj, fcarry)
                found = jnp.where(cross, 1, found)
                return acc + tot, fbin, fcarry, found
            _, fbin, fcarry, _ = lax.fori_loop(
                0, _NB // 16, sbody,
                (jnp.float32(0.0), jnp.int32(0), jnp.float32(0.0),
                 jnp.int32(0)))
            return fbin, fcarry

        bink, carryk = scan_hist(hcnt, jnp.float32(_TOPK) - jnp.float32(0.5))
        binn, carryn = scan_hist(hsum, pz)

        def p2chunk(ci, carry):
            nk0, nn0 = carry
            pltpu.sync_copy(l_hbm.at[pl.ds(row * _V + ci * _CH, _CH)], stage)

            def p2vec(vi, c2):
                nk, nn = c2
                lv = stage[pl.ds(vi * 16, 16)] / t
                key = _order_key(lv)
                biased = key ^ jnp.int32(_IMIN)
                hb = lax.shift_right_logical(biased, _SH)
                v = biased & jnp.int32((1 << _SH) - 1)
                ev = jnp.exp(lv)
                mk = hb == bink
                posk = nk + plsc.cumsum(mk.astype(jnp.int32)) - 1
                mk = jnp.logical_and(mk, posk < _BUF)
                plsc.store_scatter(bufk, [posk], v, mask=mk)
                nk = nk + jnp.max(plsc.all_reduce_population_count(mk))
                mn = hb == binn
                posn = nn + plsc.cumsum(mn.astype(jnp.int32)) - 1
                mn = jnp.logical_and(mn, posn < _BUF)
                plsc.store_scatter(bufn, [posn], v, mask=mn)
                plsc.store_scatter(bufe, [posn], ev, mask=mn)
                nn = nn + jnp.max(plsc.all_reduce_population_count(mn))
                return nk, nn
            return lax.fori_loop(0, _CH // 16, p2vec, (nk0, nn0))
        nk, nn = lax.fori_loop(0, _NCH, p2chunk,
                               (jnp.int32(0), jnp.int32(0)))

        def count_gt(buf, n, mid, weights):
            nv = (n + 15) >> 4

            def cbody(i, acc):
                chunk = buf[pl.ds(i * 16, 16)]
                valid = (i * 16 + lanes) < n
                sel = jnp.logical_and(valid, chunk > mid)
                if weights is None:
                    return acc + jnp.sum(jnp.where(sel, onesf, zerof))
                w = weights[pl.ds(i * 16, 16)]
                return acc + jnp.sum(jnp.where(sel, w, zerof))
            return lax.fori_loop(0, nv, cbody, jnp.float32(0.0))

        def bsearch(buf, n, carry, thresh, weights):
            def sbody(_, c):
                lo, hi = c
                mid = lo + ((hi - lo) >> 1)
                tot = carry + count_gt(buf, n, mid, weights)
                big = tot > thresh
                lo = jnp.where(big, mid, lo)
                hi = jnp.where(big, hi, mid)
                return lo, hi
            lo, _ = lax.fori_loop(0, _SH + 1, sbody,
                                  (jnp.int32(-1), jnp.int32((1 << _SH) - 1)))
            return lo

        lok = bsearch(bufk, nk, carryk,
                      jnp.float32(_TOPK) - jnp.float32(0.5), None)
        lon = bsearch(bufn, nn, carryn, pz, bufe)

        def to_signed(binv, lov):
            biased_thr = (binv << _SH) + lov
            sgn = biased_thr ^ jnp.int32(_IMIN)
            return jnp.where(jnp.logical_and(binv == 0, lov == -1),
                             jnp.int32(_IMIN), sgn)

        th_k = to_signed(bink, lok)
        th_n = to_signed(binn, lon)
        ob[...] = jnp.where(lanes == 0, th_k,
                            jnp.where(lanes == 1, th_n, 0))
        pltpu.sync_copy(ob, out_hbm.at[pl.ds(row * 16, 16)])


def _sc_thresholds(logits, t_arr):
    mesh = plsc.VectorSubcoreMesh(core_axis_name="c", subcore_axis_name="s")
    fn = functools.partial(
        pl.kernel, mesh=mesh,
        compiler_params=pltpu.CompilerParams(needs_layout_passes=False),
        out_type=jax.ShapeDtypeStruct((_B * 16,), jnp.int32),
        scratch_types=[
            pltpu.VMEM((_CH,), jnp.float32),
            pltpu.VMEM((_NB,), jnp.float32),
            pltpu.VMEM((_NB,), jnp.float32),
            pltpu.VMEM((_BUF,), jnp.int32),
            pltpu.VMEM((_BUF,), jnp.int32),
            pltpu.VMEM((_BUF,), jnp.float32),
            pltpu.VMEM((16,), jnp.float32),
            pltpu.VMEM((16,), jnp.int32),
        ],
    )(_sc_kernel)
    return fn(logits.reshape(-1), t_arr).reshape(_B, 16)


def _body(t_ref, th_ref, l_ref, g0_ref, g1_ref, g2_ref, g3_ref, out_ref):
    t = t_ref[0, 0]
    l = l_ref[...] / t
    key = _order_key(l)
    m = jnp.max(l, axis=-1, keepdims=True)
    e = jnp.exp(l - m)
    z = jnp.sum(e, axis=-1, keepdims=True)

    lo_k = th_ref[:, 0:1]
    lo_n = th_ref[:, 1:2]
    keep_k = key > lo_k
    keep_n = key > lo_n

    probs = e / z
    logp = jnp.log(probs + jnp.float32(1e-10))
    ent = -jnp.sum(probs * logp, axis=-1, keepdims=True)
    keep_y = jnp.abs(-logp - ent) < jnp.float32(0.5)

    neg_inf = jnp.float32(-jnp.inf)
    iota = lax.broadcasted_iota(jnp.int32, (_R, _V), 1)
    sentinel = jnp.int32(_V)

    def sample(keep, g_ref):
        g = -jnp.log(-jnp.log(g_ref[...]))
        vals = jnp.where(keep, l, neg_inf) + g
        mx = jnp.max(vals, axis=-1, keepdims=True)
        return jnp.min(jnp.where(vals == mx, iota, sentinel),
                       axis=-1, keepdims=True)

    s_n = sample(keep_n, g0_ref)
    s_k = sample(keep_k, g1_ref)
    s_t = sample(jnp.ones((_R, _V), dtype=jnp.bool_), g2_ref)
    s_y = sample(keep_y, g3_ref)
    out_ref[...] = jnp.concatenate([s_n, s_k, s_t, s_y], axis=-1)


def _mlp_body(h_ref, w1_ref, b1_ref, w2_ref, b2_ref, s_ref, out_ref):
    h = jnp.mean(h_ref[...], axis=1)
    z1 = jax.nn.relu(
        jnp.dot(h, w1_ref[...], preferred_element_type=jnp.float32)
        + b1_ref[...])
    z2 = (jnp.dot(z1, w2_ref[...], preferred_element_type=jnp.float32)
          + b2_ref[...])
    w = jax.nn.softmax(z2, axis=-1)
    samples = s_ref[...].astype(jnp.float32)
    weighted = jnp.sum(samples * w, axis=-1, keepdims=True)
    out_ref[...] = weighted.astype(jnp.int32)


@functools.partial(jax.jit, static_argnames=())
def kernel(logits, hidden_states, W1, b1, W2, b2, temperature=1.0):
    skey = jax.random.key(42)
    tiny = jnp.finfo(jnp.float32).tiny
    g = [jax.random.uniform(jax.random.fold_in(skey, i), (_B, _V), jnp.float32,
                            minval=tiny, maxval=1.0)
         for i in range(4)]
    t = jnp.asarray(temperature, jnp.float32).reshape(1, 1)
    t_arr = jnp.full((16,), jnp.asarray(temperature, jnp.float32))
    b1r = b1.reshape(1, 256)
    b2r = b2.reshape(1, 4)

    th = _sc_thresholds(logits, t_arr)

    grid = _B // _R
    row_spec = pl.BlockSpec((_R, _V), lambda i: (i, 0))
    samples = pl.pallas_call(
        _body,
        grid=(grid,),
        in_specs=[
            pl.BlockSpec((1, 1), lambda i: (0, 0)),
            pl.BlockSpec((_R, 16), lambda i: (i, 0)),
            row_spec, row_spec, row_spec, row_spec, row_spec,
        ],
        out_specs=pl.BlockSpec((_R, 4), lambda i: (i, 0)),
        out_shape=jax.ShapeDtypeStruct((_B, 4), jnp.int32),
    )(t, th, logits, g[0], g[1], g[2], g[3])

    out = pl.pallas_call(
        _mlp_body,
        in_specs=[
            pl.BlockSpec((_B, _S, _D), lambda: (0, 0, 0)),
            pl.BlockSpec((_D, 256), lambda: (0, 0)),
            pl.BlockSpec((1, 256), lambda: (0, 0)),
            pl.BlockSpec((256, 4), lambda: (0, 0)),
            pl.BlockSpec((1, 4), lambda: (0, 0)),
            pl.BlockSpec((_B, 4), lambda: (0, 0)),
        ],
        out_specs=pl.BlockSpec((_B, 1), lambda: (0, 0)),
        out_shape=jax.ShapeDtypeStruct((_B, 1), jnp.int32),
    )(hidden_states, W1, b1r, W2, b2r, samples)
    return out.reshape(_B)


# speedup vs baseline: 1.3155x; 1.1176x over previous
import functools

import jax
import jax.numpy as jnp
from jax import lax
from jax.experimental import pallas as pl
from jax.experimental.pallas import tpu as pltpu
from jax.experimental.pallas import tpu_sc as plsc

_B, _V, _S, _D = 64, 100000, 32, 768
_R = 8
_IMIN = -2147483648
_TOPK = 50
_P = 0.9

_NB = 4096
_SH = 20
_BUF = 4096
_UNR = 5


def _order_key(x):
    b = lax.bitcast_convert_type(x, jnp.int32)
    return jnp.where(b >= 0, b, jnp.int32(_IMIN) - b)


def _sc_kernel(l_hbm, t_hbm, out_hbm, stage, hcnt, hsum, bufk, bufn, bufe,
               tv, ob):
    wid = lax.axis_index("s") * 2 + lax.axis_index("c")
    pltpu.sync_copy(t_hbm, tv)
    t = tv[...]
    lanes = lax.iota(jnp.int32, 16)
    onesf = jnp.ones((16,), jnp.float32)
    zerof = jnp.zeros((16,), jnp.float32)

    for rr in range(2):
        row = wid * 2 + rr

        def zbody(i, c):
            hcnt[pl.ds(i * 16, 16)] = zerof
            hsum[pl.ds(i * 16, 16)] = zerof
            return c
        lax.fori_loop(0, _NB // 16, zbody, 0)

        pltpu.sync_copy(l_hbm.at[pl.ds(row * _V, _V)], stage)

        def p1vec(vi, c2):
            for u in range(_UNR):
                lv = stage[pl.ds((vi * _UNR + u) * 16, 16)] / t
                key = _order_key(lv)
                biased = key ^ jnp.int32(_IMIN)
                hb = lax.shift_right_logical(biased, _SH)
                ev = jnp.exp(lv)
                plsc.addupdate_scatter(hcnt, [hb], onesf)
                plsc.addupdate_scatter(hsum, [hb], ev)
            return c2
        lax.fori_loop(0, _V // 16 // _UNR, p1vec, 0)

        def zsum(i, acc):
            return acc + jnp.sum(hsum[pl.ds(i * 16, 16)])
        ztot = lax.fori_loop(0, _NB // 16, zsum, jnp.float32(0.0))
        pz = jnp.float32(_P) * ztot

        def scan_hist(href, thresh):
            def sbody(i, carry):
                acc, fbin, fcarry, found = carry
                c = (_NB // 16 - 1) - i
                vec = href[pl.ds(c * 16, 16)]
                tot = jnp.sum(vec)
                cross = jnp.logical_and(found == 0, acc + tot > thresh)
                d = lax.rev(vec, (0,))
                cum = plsc.cumsum(d)
                mask = (acc + cum) > thresh
                j = jnp.max(plsc.all_reduce_ffs(mask))
                cumprev = cum - d
                catj = acc + jnp.sum(jnp.where(lanes == j, cumprev, zerof))
                binj = c * 16 + 15 - j
                fbin = jnp.where(cross, binj, fbin)
                fcarry = jnp.where(cross, catj, fcarry)
                found = jnp.where(cross, 1, found)
                return acc + tot, fbin, fcarry, found
            _, fbin, fcarry, _ = lax.fori_loop(
                0, _NB // 16, sbody,
                (jnp.float32(0.0), jnp.int32(0), jnp.float32(0.0),
                 jnp.int32(0)))
            return fbin, fcarry

        bink, carryk = scan_hist(hcnt, jnp.float32(_TOPK) - jnp.float32(0.5))
        binn, carryn = scan_hist(hsum, pz)

        def p2vec(vi, c2):
            nk, nn = c2
            for u in range(_UNR):
                lv = stage[pl.ds((vi * _UNR + u) * 16, 16)] / t
                key = _order_key(lv)
                biased = key ^ jnp.int32(_IMIN)
                hb = lax.shift_right_logical(biased, _SH)
                v = biased & jnp.int32((1 << _SH) - 1)
                ev = jnp.exp(lv)
                mk = hb == bink
                posk = nk + plsc.cumsum(mk.astype(jnp.int32)) - 1
                mk = jnp.logical_and(mk, posk < _BUF)
                plsc.store_scatter(bufk, [posk], v, mask=mk)
                nk = nk + jnp.max(plsc.all_reduce_population_count(mk))
                mn = hb == binn
                posn = nn + plsc.cumsum(mn.astype(jnp.int32)) - 1
                mn = jnp.logical_and(mn, posn < _BUF)
                plsc.store_scatter(bufn, [posn], v, mask=mn)
                plsc.store_scatter(bufe, [posn], ev, mask=mn)
                nn = nn + jnp.max(plsc.all_reduce_population_count(mn))
            return nk, nn
        nk, nn = lax.fori_loop(0, _V // 16 // _UNR, p2vec,
                               (jnp.int32(0), jnp.int32(0)))

        def count_gt(buf, n, mid, weights):
            nv = (n + 15) >> 4

            def cbody(i, acc):
                chunk = buf[pl.ds(i * 16, 16)]
                valid = (i * 16 + lanes) < n
                sel = jnp.logical_and(valid, chunk > mid)
                if weights is None:
                    return acc + jnp.sum(jnp.where(sel, onesf, zerof))
                w = weights[pl.ds(i * 16, 16)]
                return acc + jnp.sum(jnp.where(sel, w, zerof))
            return lax.fori_loop(0, nv, cbody, jnp.float32(0.0))

        def bsearch(buf, n, carry, thresh, weights):
            def sbody(_, c):
                lo, hi = c
                mid = lo + ((hi - lo) >> 1)
                tot = carry + count_gt(buf, n, mid, weights)
                big = tot > thresh
                lo = jnp.where(big, mid, lo)
                hi = jnp.where(big, hi, mid)
                return lo, hi
            lo, _ = lax.fori_loop(0, _SH + 1, sbody,
                                  (jnp.int32(-1), jnp.int32((1 << _SH) - 1)))
            return lo

        lok = bsearch(bufk, nk, carryk,
                      jnp.float32(_TOPK) - jnp.float32(0.5), None)
        lon = bsearch(bufn, nn, carryn, pz, bufe)

        def to_signed(binv, lov):
            biased_thr = (binv << _SH) + lov
            sgn = biased_thr ^ jnp.int32(_IMIN)
            return jnp.where(jnp.logical_and(binv == 0, lov == -1),
                             jnp.int32(_IMIN), sgn)

        th_k = to_signed(bink, lok)
        th_n = to_signed(binn, lon)
        ob[...] = jnp.where(lanes == 0, th_k,
                            jnp.where(lanes == 1, th_n, 0))
        pltpu.sync_copy(ob, out_hbm.at[pl.ds(row * 16, 16)])


def _sc_thresholds(logits, t_arr):
    mesh = plsc.VectorSubcoreMesh(core_axis_name="c", subcore_axis_name="s")
    fn = functools.partial(
        pl.kernel, mesh=mesh,
        compiler_params=pltpu.CompilerParams(needs_layout_passes=False),
        out_type=jax.ShapeDtypeStruct((_B * 16,), jnp.int32),
        scratch_types=[
            pltpu.VMEM((_V,), jnp.float32),
            pltpu.VMEM((_NB,), jnp.float32),
            pltpu.VMEM((_NB,), jnp.float32),
            pltpu.VMEM((_BUF,), jnp.int32),
            pltpu.VMEM((_BUF,), jnp.int32),
            pltpu.VMEM((_BUF,), jnp.float32),
            pltpu.VMEM((16,), jnp.float32),
            pltpu.VMEM((16,), jnp.int32),
        ],
    )(_sc_kernel)
    return fn(logits.reshape(-1), t_arr).reshape(_B, 16)


def _body(t_ref, th_ref, l_ref, g0_ref, g1_ref, g2_ref, g3_ref, out_ref):
    t = t_ref[0, 0]
    l = l_ref[...] / t
    key = _order_key(l)
    m = jnp.max(l, axis=-1, keepdims=True)
    e = jnp.exp(l - m)
    z = jnp.sum(e, axis=-1, keepdims=True)

    lo_k = th_ref[:, 0:1]
    lo_n = th_ref[:, 1:2]
    keep_k = key > lo_k
    keep_n = key > lo_n

    probs = e / z
    logp = jnp.log(probs + jnp.float32(1e-10))
    ent = -jnp.sum(probs * logp, axis=-1, keepdims=True)
    keep_y = jnp.abs(-logp - ent) < jnp.float32(0.5)

    neg_inf = jnp.float32(-jnp.inf)
    iota = lax.broadcasted_iota(jnp.int32, (_R, _V), 1)
    sentinel = jnp.int32(_V)

    def sample(keep, g_ref):
        g = -jnp.log(-jnp.log(g_ref[...]))
        vals = jnp.where(keep, l, neg_inf) + g
        mx = jnp.max(vals, axis=-1, keepdims=True)
        return jnp.min(jnp.where(vals == mx, iota, sentinel),
                       axis=-1, keepdims=True)

    s_n = sample(keep_n, g0_ref)
    s_k = sample(keep_k, g1_ref)
    s_t = sample(jnp.ones((_R, _V), dtype=jnp.bool_), g2_ref)
    s_y = sample(keep_y, g3_ref)
    out_ref[...] = jnp.concatenate([s_n, s_k, s_t, s_y], axis=-1)


def _mlp_body(h_ref, w1_ref, b1_ref, w2_ref, b2_ref, s_ref, out_ref):
    h = jnp.mean(h_ref[...], axis=1)
    z1 = jax.nn.relu(
        jnp.dot(h, w1_ref[...], preferred_element_type=jnp.float32)
        + b1_ref[...])
    z2 = (jnp.dot(z1, w2_ref[...], preferred_element_type=jnp.float32)
          + b2_ref[...])
    w = jax.nn.softmax(z2, axis=-1)
    samples = s_ref[...].astype(jnp.float32)
    weighted = jnp.sum(samples * w, axis=-1, keepdims=True)
    out_ref[...] = weighted.astype(jnp.int32)


@functools.partial(jax.jit, static_argnames=())
def kernel(logits, hidden_states, W1, b1, W2, b2, temperature=1.0):
    skey = jax.random.key(42)
    tiny = jnp.finfo(jnp.float32).tiny
    g = [jax.random.uniform(jax.random.fold_in(skey, i), (_B, _V), jnp.float32,
                            minval=tiny, maxval=1.0)
         for i in range(4)]
    t = jnp.asarray(temperature, jnp.float32).reshape(1, 1)
    t_arr = jnp.full((16,), jnp.asarray(temperature, jnp.float32))
    b1r = b1.reshape(1, 256)
    b2r = b2.reshape(1, 4)

    th = _sc_thresholds(logits, t_arr)

    grid = _B // _R
    row_spec = pl.BlockSpec((_R, _V), lambda i: (i, 0))
    samples = pl.pallas_call(
        _body,
        grid=(grid,),
        in_specs=[
            pl.BlockSpec((1, 1), lambda i: (0, 0)),
            pl.BlockSpec((_R, 16), lambda i: (i, 0)),
            row_spec, row_spec, row_spec, row_spec, row_spec,
        ],
        out_specs=pl.BlockSpec((_R, 4), lambda i: (i, 0)),
        out_shape=jax.ShapeDtypeStruct((_B, 4), jnp.int32),
    )(t, th, logits, g[0], g[1], g[2], g[3])

    out = pl.pallas_call(
        _mlp_body,
        in_specs=[
            pl.BlockSpec((_B, _S, _D), lambda: (0, 0, 0)),
            pl.BlockSpec((_D, 256), lambda: (0, 0)),
            pl.BlockSpec((1, 256), lambda: (0, 0)),
            pl.BlockSpec((256, 4), lambda: (0, 0)),
            pl.BlockSpec((1, 4), lambda: (0, 0)),
            pl.BlockSpec((_B, 4), lambda: (0, 0)),
        ],
        out_specs=pl.BlockSpec((_B, 1), lambda: (0, 0)),
        out_shape=jax.ShapeDtypeStruct((_B, 1), jnp.int32),
    )(hidden_states, W1, b1r, W2, b2r, samples)
    return out.reshape(_B)


# speedup vs baseline: 1.3172x; 1.0013x over previous
import functools

import jax
import jax.numpy as jnp
from jax import lax
from jax.experimental import pallas as pl
from jax.experimental.pallas import tpu as pltpu
from jax.experimental.pallas import tpu_sc as plsc

_B, _V, _S, _D = 64, 100000, 32, 768
_R = 8
_IMIN = -2147483648
_TOPK = 50
_P = 0.9

_NB = 4096
_SH = 20
_BUF = 4096
_UNR = 10


def _order_key(x):
    b = lax.bitcast_convert_type(x, jnp.int32)
    return jnp.where(b >= 0, b, jnp.int32(_IMIN) - b)


def _sc_kernel(l_hbm, t_hbm, out_hbm, stage, hcnt, hsum, bufk, bufn, bufe,
               tv, ob):
    wid = lax.axis_index("s") * 2 + lax.axis_index("c")
    pltpu.sync_copy(t_hbm, tv)
    t = tv[...]
    lanes = lax.iota(jnp.int32, 16)
    onesf = jnp.ones((16,), jnp.float32)
    zerof = jnp.zeros((16,), jnp.float32)

    for rr in range(2):
        row = wid * 2 + rr

        def zbody(i, c):
            hcnt[pl.ds(i * 16, 16)] = zerof
            hsum[pl.ds(i * 16, 16)] = zerof
            return c
        lax.fori_loop(0, _NB // 16, zbody, 0)

        pltpu.sync_copy(l_hbm.at[pl.ds(row * _V, _V)], stage)

        def p1vec(vi, c2):
            for u in range(_UNR):
                lv = stage[pl.ds((vi * _UNR + u) * 16, 16)] / t
                key = _order_key(lv)
                biased = key ^ jnp.int32(_IMIN)
                hb = lax.shift_right_logical(biased, _SH)
                ev = jnp.exp(lv)
                plsc.addupdate_scatter(hcnt, [hb], onesf)
                plsc.addupdate_scatter(hsum, [hb], ev)
            return c2
        lax.fori_loop(0, _V // 16 // _UNR, p1vec, 0)

        def zsum(i, acc):
            return acc + jnp.sum(hsum[pl.ds(i * 16, 16)])
        ztot = lax.fori_loop(0, _NB // 16, zsum, jnp.float32(0.0))
        pz = jnp.float32(_P) * ztot

        def scan_hist(href, thresh):
            def sbody(i, carry):
                acc, fbin, fcarry, found = carry
                c = (_NB // 16 - 1) - i
                vec = href[pl.ds(c * 16, 16)]
                tot = jnp.sum(vec)
                cross = jnp.logical_and(found == 0, acc + tot > thresh)
                d = lax.rev(vec, (0,))
                cum = plsc.cumsum(d)
                mask = (acc + cum) > thresh
                j = jnp.max(plsc.all_reduce_ffs(mask))
                cumprev = cum - d
                catj = acc + jnp.sum(jnp.where(lanes == j, cumprev, zerof))
                binj = c * 16 + 15 - j
                fbin = jnp.where(cross, binj, fbin)
                fcarry = jnp.where(cross, catj, fcarry)
                found = jnp.where(cross, 1, found)
                return acc + tot, fbin, fcarry, found
            _, fbin, fcarry, _ = lax.fori_loop(
                0, _NB // 16, sbody,
                (jnp.float32(0.0), jnp.int32(0), jnp.float32(0.0),
                 jnp.int32(0)))
            return fbin, fcarry

        bink, carryk = scan_hist(hcnt, jnp.float32(_TOPK) - jnp.float32(0.5))
        binn, carryn = scan_hist(hsum, pz)

        def p2vec(vi, c2):
            nk, nn = c2
            for u in range(_UNR):
                lv = stage[pl.ds((vi * _UNR + u) * 16, 16)] / t
                key = _order_key(lv)
                biased = key ^ jnp.int32(_IMIN)
                hb = lax.shift_right_logical(biased, _SH)
                v = biased & jnp.int32((1 << _SH) - 1)
                ev = jnp.exp(lv)
                mk = hb == bink
                posk = nk + plsc.cumsum(mk.astype(jnp.int32)) - 1
                mk = jnp.logical_and(mk, posk < _BUF)
                plsc.store_scatter(bufk, [posk], v, mask=mk)
                nk = nk + jnp.max(plsc.all_reduce_population_count(mk))
                mn = hb == binn
                posn = nn + plsc.cumsum(mn.astype(jnp.int32)) - 1
                mn = jnp.logical_and(mn, posn < _BUF)
                plsc.store_scatter(bufn, [posn], v, mask=mn)
                plsc.store_scatter(bufe, [posn], ev, mask=mn)
                nn = nn + jnp.max(plsc.all_reduce_population_count(mn))
            return nk, nn
        nk, nn = lax.fori_loop(0, _V // 16 // _UNR, p2vec,
                               (jnp.int32(0), jnp.int32(0)))

        def count_gt(buf, n, mid, weights):
            nv = (n + 15) >> 4

            def cbody(i, acc):
                chunk = buf[pl.ds(i * 16, 16)]
                valid = (i * 16 + lanes) < n
                sel = jnp.logical_and(valid, chunk > mid)
                if weights is None:
                    return acc + jnp.sum(jnp.where(sel, onesf, zerof))
                w = weights[pl.ds(i * 16, 16)]
                return acc + jnp.sum(jnp.where(sel, w, zerof))
            return lax.fori_loop(0, nv, cbody, jnp.float32(0.0))

        def bsearch(buf, n, carry, thresh, weights):
            def sbody(_, c):
                lo, hi = c
                mid = lo + ((hi - lo) >> 1)
                tot = carry + count_gt(buf, n, mid, weights)
                big = tot > thresh
                lo = jnp.where(big, mid, lo)
                hi = jnp.where(big, hi, mid)
                return lo, hi
            lo, _ = lax.fori_loop(0, _SH + 1, sbody,
                                  (jnp.int32(-1), jnp.int32((1 << _SH) - 1)))
            return lo

        lok = bsearch(bufk, nk, carryk,
                      jnp.float32(_TOPK) - jnp.float32(0.5), None)
        lon = bsearch(bufn, nn, carryn, pz, bufe)

        def to_signed(binv, lov):
            biased_thr = (binv << _SH) + lov
            sgn = biased_thr ^ jnp.int32(_IMIN)
            return jnp.where(jnp.logical_and(binv == 0, lov == -1),
                             jnp.int32(_IMIN), sgn)

        th_k = to_signed(bink, lok)
        th_n = to_signed(binn, lon)
        ob[...] = jnp.where(lanes == 0, th_k,
                            jnp.where(lanes == 1, th_n, 0))
        pltpu.sync_copy(ob, out_hbm.at[pl.ds(row * 16, 16)])


def _sc_thresholds(logits, t_arr):
    mesh = plsc.VectorSubcoreMesh(core_axis_name="c", subcore_axis_name="s")
    fn = functools.partial(
        pl.kernel, mesh=mesh,
        compiler_params=pltpu.CompilerParams(needs_layout_passes=False),
        out_type=jax.ShapeDtypeStruct((_B * 16,), jnp.int32),
        scratch_types=[
            pltpu.VMEM((_V,), jnp.float32),
            pltpu.VMEM((_NB,), jnp.float32),
            pltpu.VMEM((_NB,), jnp.float32),
            pltpu.VMEM((_BUF,), jnp.int32),
            pltpu.VMEM((_BUF,), jnp.int32),
            pltpu.VMEM((_BUF,), jnp.float32),
            pltpu.VMEM((16,), jnp.float32),
            pltpu.VMEM((16,), jnp.int32),
        ],
    )(_sc_kernel)
    return fn(logits.reshape(-1), t_arr).reshape(_B, 16)


def _body(t_ref, th_ref, l_ref, g0_ref, g1_ref, g2_ref, g3_ref, out_ref):
    t = t_ref[0, 0]
    l = l_ref[...] / t
    key = _order_key(l)
    m = jnp.max(l, axis=-1, keepdims=True)
    e = jnp.exp(l - m)
    z = jnp.sum(e, axis=-1, keepdims=True)

    lo_k = th_ref[:, 0:1]
    lo_n = th_ref[:, 1:2]
    keep_k = key > lo_k
    keep_n = key > lo_n

    probs = e / z
    logp = jnp.log(probs + jnp.float32(1e-10))
    ent = -jnp.sum(probs * logp, axis=-1, keepdims=True)
    keep_y = jnp.abs(-logp - ent) < jnp.float32(0.5)

    neg_inf = jnp.float32(-jnp.inf)
    iota = lax.broadcasted_iota(jnp.int32, (_R, _V), 1)
    sentinel = jnp.int32(_V)

    def sample(keep, g_ref):
        g = -jnp.log(-jnp.log(g_ref[...]))
        vals = jnp.where(keep, l, neg_inf) + g
        mx = jnp.max(vals, axis=-1, keepdims=True)
        return jnp.min(jnp.where(vals == mx, iota, sentinel),
                       axis=-1, keepdims=True)

    s_n = sample(keep_n, g0_ref)
    s_k = sample(keep_k, g1_ref)
    s_t = sample(jnp.ones((_R, _V), dtype=jnp.bool_), g2_ref)
    s_y = sample(keep_y, g3_ref)
    out_ref[...] = jnp.concatenate([s_n, s_k, s_t, s_y], axis=-1)


def _mlp_body(h_ref, w1_ref, b1_ref, w2_ref, b2_ref, s_ref, out_ref):
    h = jnp.mean(h_ref[...], axis=1)
    z1 = jax.nn.relu(
        jnp.dot(h, w1_ref[...], preferred_element_type=jnp.float32)
        + b1_ref[...])
    z2 = (jnp.dot(z1, w2_ref[...], preferred_element_type=jnp.float32)
          + b2_ref[...])
    w = jax.nn.softmax(z2, axis=-1)
    samples = s_ref[...].astype(jnp.float32)
    weighted = jnp.sum(samples * w, axis=-1, keepdims=True)
    out_ref[...] = weighted.astype(jnp.int32)


@functools.partial(jax.jit, static_argnames=())
def kernel(logits, hidden_states, W1, b1, W2, b2, temperature=1.0):
    skey = jax.random.key(42)
    tiny = jnp.finfo(jnp.float32).tiny
    g = [jax.random.uniform(jax.random.fold_in(skey, i), (_B, _V), jnp.float32,
                            minval=tiny, maxval=1.0)
         for i in range(4)]
    t = jnp.asarray(temperature, jnp.float32).reshape(1, 1)
    t_arr = jnp.full((16,), jnp.asarray(temperature, jnp.float32))
    b1r = b1.reshape(1, 256)
    b2r = b2.reshape(1, 4)

    th = _sc_thresholds(logits, t_arr)

    grid = _B // _R
    row_spec = pl.BlockSpec((_R, _V), lambda i: (i, 0))
    samples = pl.pallas_call(
        _body,
        grid=(grid,),
        in_specs=[
            pl.BlockSpec((1, 1), lambda i: (0, 0)),
            pl.BlockSpec((_R, 16), lambda i: (i, 0)),
            row_spec, row_spec, row_spec, row_spec, row_spec,
        ],
        out_specs=pl.BlockSpec((_R, 4), lambda i: (i, 0)),
        out_shape=jax.ShapeDtypeStruct((_B, 4), jnp.int32),
    )(t, th, logits, g[0], g[1], g[2], g[3])

    out = pl.pallas_call(
        _mlp_body,
        in_specs=[
            pl.BlockSpec((_B, _S, _D), lambda: (0, 0, 0)),
            pl.BlockSpec((_D, 256), lambda: (0, 0)),
            pl.BlockSpec((1, 256), lambda: (0, 0)),
            pl.BlockSpec((256, 4), lambda: (0, 0)),
            pl.BlockSpec((1, 4), lambda: (0, 0)),
            pl.BlockSpec((_B, 4), lambda: (0, 0)),
        ],
        out_specs=pl.BlockSpec((_B, 1), lambda: (0, 0)),
        out_shape=jax.ShapeDtypeStruct((_B, 1), jnp.int32),
    )(hidden_states, W1, b1r, W2, b2r, samples)
    return out.reshape(_B)
